# baseline (speedup 1.0000x reference)
import functools
import math

import jax
import jax.numpy as jnp
from jax import lax
from jax.experimental import pallas as pl
from jax.experimental.pallas import tpu as pltpu
from jax.experimental.pallas import tpu_sc as plsc

C = 72
CP = 128
PE = 8
H = 4
DH = 18
G = 50
NPG = 200
N = G * NPG
NPAD = 10240
E = 320000
NW = 32
K = 80
ROWS_W = (E // K) // NW
ROWS_T = NPAD // 16


def _edge_body(h_hbm, pk_hbm, eemb_hbm, out_hbm,
               pk_v, srcc, dstc, eac, rows, erows, zbuf, shared,
               sem_h, sem_e):
    cid = lax.axis_index("c")
    sid = lax.axis_index("s")
    wid = sid * 2 + cid

    def zrow(i, _):
        for j in range(CP // 16):
            zbuf[i, pl.ds(j * 16, 16)] = jnp.zeros((16,), jnp.float32)
        return 0
    lax.fori_loop(0, K, zrow, 0)

    def zslice(z, _):
        pltpu.sync_copy(zbuf, shared.at[pl.ds(sid * ROWS_T + z * K, K)])
        return 0
    lax.fori_loop(0, ROWS_T // K, zslice, 0)
    plsc.subcore_barrier()

    pltpu.sync_copy(pk_hbm.at[wid], pk_v)

    def chunk(g, _):
        for j in range(K // 16):
            sl = pl.ds(j * 16, 16)
            w = pk_v[g, sl]
            srcc[sl] = w & 0x3FFF
            dstc[sl] = (w >> 14) & 0x3FFF
            eac[sl] = (w >> 28) & 3
        cp1 = pltpu.async_copy(h_hbm.at[srcc], rows, sem_h)
        cp2 = pltpu.async_copy(eemb_hbm.at[eac], erows, sem_e)
        cp1.wait()
        cp2.wait()

        def body(i, _):
            for j in range(CP // 16):
                sl = pl.ds(j * 16, 16)
                rows[i, sl] = jnp.maximum(rows[i, sl] + erows[i, sl], 0.0)
            return 0
        lax.fori_loop(0, K, body, 0)
        pltpu.sync_copy(rows, shared.at[dstc], add=True)
        return 0

    lax.fori_loop(0, ROWS_W, chunk, 0)
    plsc.subcore_barrier()
    pltpu.sync_copy(shared.at[pl.ds(sid * ROWS_T, ROWS_T)],
                    out_hbm.at[pl.ds(cid * NPAD + sid * ROWS_T, ROWS_T)])


@functools.cache
def _get_edge_call():
    return pl.kernel(
        _edge_body,
        out_type=jax.ShapeDtypeStruct((2 * NPAD, CP), jnp.float32),
        mesh=plsc.VectorSubcoreMesh(core_axis_name="c", subcore_axis_name="s"),
        scratch_types=[
            pltpu.VMEM((ROWS_W, K), jnp.int32),
            pltpu.VMEM((K,), jnp.int32),
            pltpu.VMEM((K,), jnp.int32),
            pltpu.VMEM((K,), jnp.int32),
            pltpu.VMEM((K, CP), jnp.float32),
            pltpu.VMEM((K, CP), jnp.float32),
            pltpu.VMEM((K, CP), jnp.float32),
            pltpu.VMEM_SHARED((NPAD, CP), jnp.float32),
            pltpu.SemaphoreType.DMA,
            pltpu.SemaphoreType.DMA,
        ],
        compiler_params=pltpu.CompilerParams(use_tc_tiling_on_sc=False),
    )


def _embed_body(x_ref, pe_ref, nemb_ref, plw_ref, plb_ref, pbg_ref, pbb_ref,
                out_ref):
    x = x_ref[...]
    pe = pe_ref[...]
    onehot = (x == lax.broadcasted_iota(jnp.int32, (N, 28), 1)
              ).astype(jnp.float32)
    emb = jnp.dot(onehot, nemb_ref[...],
                  preferred_element_type=jnp.float32)
    m = jnp.mean(pe, axis=0, keepdims=True)
    v = jnp.mean((pe - m) * (pe - m), axis=0, keepdims=True)
    pe_n = (pe - m) * lax.rsqrt(v + 1e-5) * pbg_ref[...] + pbb_ref[...]
    pe_p = jnp.dot(pe_n, plw_ref[...],
                   preferred_element_type=jnp.float32) + plb_ref[...]
    zeros = jnp.zeros((N, CP - C), jnp.float32)
    out_ref[...] = jnp.concatenate([emb, pe_p, zeros], axis=1)


def _embed(x, pe, node_emb, pe_lin_w, pe_lin_b, pe_bn_g, pe_bn_b):
    return pl.pallas_call(
        _embed_body,
        out_shape=jax.ShapeDtypeStruct((N, CP), jnp.float32),
    )(x.reshape(N, 1).astype(jnp.int32), pe, node_emb,
      pe_lin_w, pe_lin_b.reshape(1, PE),
      pe_bn_g.reshape(1, PE), pe_bn_b.reshape(1, PE))


def _bn(x, g, b):
    m = jnp.mean(x, axis=0, keepdims=True)
    v = jnp.mean((x - m) * (x - m), axis=0, keepdims=True)
    return (x - m) * lax.rsqrt(v + 1e-5) * g + b


def _layer_body(h_ref, p0_ref, p1_ref,
                gw1_ref, gb1_ref, gw2_ref, gb2_ref,
                wq_ref, wk_ref, wv_ref, wo_ref,
                bq_ref, bk_ref, bv_ref, bo_ref,
                b1g_ref, b1b_ref, b2g_ref, b2b_ref, b3g_ref, b3b_ref,
                mw1_ref, mb1_ref, mw2_ref, mb2_ref,
                out_ref, ao_scr):
    h = h_ref[...][:, :C]
    aggr = (p0_ref[...][:N, :C] + p1_ref[...][:N, :C])

    t = h + aggr
    t = jnp.maximum(jnp.dot(t, gw1_ref[...],
                            preferred_element_type=jnp.float32)
                    + gb1_ref[...], 0.0)
    t = jnp.dot(t, gw2_ref[...],
                preferred_element_type=jnp.float32) + gb2_ref[...]
    hl = _bn(t + h, b1g_ref[...], b1b_ref[...])

    inv_sqrt = 1.0 / math.sqrt(float(DH))

    def graph(g, _):
        base = g * NPG
        hg = h_ref[pl.ds(base, NPG), :][:, :C]
        acc = jnp.zeros((NPG, C), jnp.float32)
        for hh in range(H):
            qg = jnp.dot(hg, wq_ref[hh],
                         preferred_element_type=jnp.float32) + bq_ref[hh]
            kg = jnp.dot(hg, wk_ref[hh],
                         preferred_element_type=jnp.float32) + bk_ref[hh]
            vg = jnp.dot(hg, wv_ref[hh],
                         preferred_element_type=jnp.float32) + bv_ref[hh]
            s = lax.dot_general(qg, kg, (((1,), (1,)), ((), ())),
                                preferred_element_type=jnp.float32)
            s = s * inv_sqrt
            s = s - jnp.max(s, axis=1, keepdims=True)
            p = jnp.exp(s)
            p = p / jnp.sum(p, axis=1, keepdims=True)
            og = jnp.dot(p, vg, preferred_element_type=jnp.float32)
            acc = acc + jnp.dot(og, wo_ref[hh],
                                preferred_element_type=jnp.float32)
        ao_scr[pl.ds(base, NPG), :] = acc
        return 0

    lax.fori_loop(0, G, graph, 0)

    ao = ao_scr[...] + bo_ref[...]
    ha = _bn(ao + h, b2g_ref[...], b2b_ref[...])

    out = hl + ha
    ff = jnp.maximum(jnp.dot(out, mw1_ref[...],
                             preferred_element_type=jnp.float32)
                     + mb1_ref[...], 0.0)
    ff = jnp.dot(ff, mw2_ref[...],
                 preferred_element_type=jnp.float32) + mb2_ref[...]
    hn = _bn(out + ff, b3g_ref[...], b3b_ref[...])
    out_ref[...] = jnp.concatenate(
        [hn, jnp.zeros((N, CP - C), jnp.float32)], axis=1)


def _layer(h, parts, w):
    return pl.pallas_call(
        _layer_body,
        out_shape=jax.ShapeDtypeStruct((N, CP), jnp.float32),
        scratch_shapes=[
            pltpu.VMEM((N, C), jnp.float32),
        ],
    )(h, parts[:NPAD], parts[NPAD:], *w)


def _readout_body(h1_ref, h2_ref, ow_ref, ob_ref, out_ref):
    seg = (lax.broadcasted_iota(jnp.int32, (G, N), 1) // NPG
           == lax.broadcasted_iota(jnp.int32, (G, N), 0)
           ).astype(jnp.float32)
    g1 = jnp.dot(seg, h1_ref[...][:, :C],
                 preferred_element_type=jnp.float32)
    g2 = jnp.dot(seg, h2_ref[...][:, :C],
                 preferred_element_type=jnp.float32)
    g1 = jnp.dot(g1, ow_ref[...],
                 preferred_element_type=jnp.float32) + ob_ref[...]
    g2 = jnp.dot(g2, ow_ref[...],
                 preferred_element_type=jnp.float32) + ob_ref[...]
    n1 = jnp.maximum(jnp.sqrt(jnp.sum(g1 * g1, axis=1, keepdims=True)), 1e-8)
    n2 = jnp.maximum(jnp.sqrt(jnp.sum(g2 * g2, axis=1, keepdims=True)), 1e-8)
    dot = jnp.sum(g1 * g2, axis=1, keepdims=True)
    out_ref[...] = dot / (n1 * n2)


def _readout(h1, h2, out_w, out_b):
    r = pl.pallas_call(
        _readout_body,
        out_shape=jax.ShapeDtypeStruct((G, 1), jnp.float32),
    )(h1, h2, out_w, out_b.reshape(1, 64))
    return r.reshape(G)


def kernel(x1, pe1, edge_index1, edge_attr1, batch1,
           x2, pe2, edge_index2, edge_attr2, batch2,
           node_emb, pe_lin_w, pe_lin_b, pe_bn_g, pe_bn_b, edge_emb,
           gine_w1, gine_b1, gine_w2, gine_b2,
           wq, wk, wv, wo, bq, bk, bv, bo,
           bn1_g, bn1_b, bn2_g, bn2_b, bn3_g, bn3_b,
           mlp_w1, mlp_b1, mlp_w2, mlp_b2, out_w, out_b):
    L = 2
    eemb_pad = jnp.concatenate(
        [edge_emb, jnp.zeros((4, CP - C), jnp.float32)], axis=1)

    def edge_pack(ei, ea):
        src = ei[0].astype(jnp.int32)
        dst = ei[1].astype(jnp.int32)
        eav = ea.astype(jnp.int32)
        return (src | (dst << 14) | (eav << 28)).reshape(NW, ROWS_W, K)

    pk1 = edge_pack(edge_index1, edge_attr1)
    pk2 = edge_pack(edge_index2, edge_attr2)

    def layer_weights(l):
        def heads(wm, bm):
            return (wm[l].reshape(C, H, DH).transpose(1, 0, 2),
                    bm[l].reshape(H, 1, DH))
        wqh, bqh = heads(wq, bq)
        wkh, bkh = heads(wk, bk)
        wvh, bvh = heads(wv, bv)
        woh = wo[l].reshape(H, DH, C)
        return (gine_w1[l], gine_b1[l].reshape(1, C),
                gine_w2[l], gine_b2[l].reshape(1, C),
                wqh, wkh, wvh, woh,
                bqh, bkh, bvh, bo[l].reshape(1, C),
                bn1_g[l].reshape(1, C), bn1_b[l].reshape(1, C),
                bn2_g[l].reshape(1, C), bn2_b[l].reshape(1, C),
                bn3_g[l].reshape(1, C), bn3_b[l].reshape(1, C),
                mlp_w1[l], mlp_b1[l].reshape(1, 2 * C),
                mlp_w2[l], mlp_b2[l].reshape(1, C))

    h1 = _embed(x1, pe1, node_emb, pe_lin_w, pe_lin_b, pe_bn_g, pe_bn_b)
    h2 = _embed(x2, pe2, node_emb, pe_lin_w, pe_lin_b, pe_bn_g, pe_bn_b)

    edge_call = _get_edge_call()
    for l in range(L):
        w = layer_weights(l)
        parts1 = edge_call(h1, pk1, eemb_pad)
        h1 = _layer(h1, parts1, w)
        parts2 = edge_call(h2, pk2, eemb_pad)
        h2 = _layer(h2, parts2, w)

    return _readout(h1, h2, out_w, out_b)


# speedup vs baseline: 14.1913x; 14.1913x over previous
import functools
import math

import jax
import jax.numpy as jnp
from jax import lax
from jax.experimental import pallas as pl
from jax.experimental.pallas import tpu as pltpu
from jax.experimental.pallas import tpu_sc as plsc

C = 72
CP = 128
PE = 8
H = 4
DH = 18
G = 50
NPG = 200
N = G * NPG
NPAD = 10240
E = 320000
NW = 32
K = 80
CHUNKS = (E // NW) // K
ROWS_T = NPAD // 16
ZR = 80


def _edge_body(tab_hbm, pk_hbm, out_hbm,
               pk_v, g0, d0, g1, d1, rows0, rows1, shared,
               sem0, sem1):
    cid = lax.axis_index("c")
    sid = lax.axis_index("s")
    wid = sid * 2 + cid

    def zrow(i, _):
        for j in range(CP // 16):
            rows0[i, pl.ds(j * 16, 16)] = jnp.zeros((16,), jnp.float32)
        return 0
    lax.fori_loop(0, ZR, zrow, 0)
    for z in range(ROWS_T // ZR):
        pltpu.sync_copy(rows0.at[pl.ds(0, ZR)],
                        shared.at[pl.ds(sid * ROWS_T + z * ZR, ZR)])
    plsc.subcore_barrier()

    pltpu.sync_copy(pk_hbm.at[wid], pk_v)

    def unpack(c, gbuf, dbuf):
        for j in range(K // 16):
            sl = pl.ds(j * 16, 16)
            w = pk_v[c, sl]
            gbuf[sl] = w & 0xFFFF
            dbuf[sl] = (w >> 16) & 0x3FFF

    unpack(0, g0, d0)
    pltpu.async_copy(tab_hbm.at[g0], rows0, sem0)

    def pair(i, _):
        g = i * 2
        unpack(g + 1, g1, d1)
        pltpu.async_copy(tab_hbm.at[g1], rows1, sem1)
        pltpu.make_async_copy(tab_hbm.at[g0], rows0, sem0).wait()
        pltpu.sync_copy(rows0, shared.at[d0], add=True)

        @pl.when(g + 2 < CHUNKS)
        def _():
            unpack(g + 2, g0, d0)
            pltpu.async_copy(tab_hbm.at[g0], rows0, sem0)

        pltpu.make_async_copy(tab_hbm.at[g1], rows1, sem1).wait()
        pltpu.sync_copy(rows1, shared.at[d1], add=True)
        return 0

    lax.fori_loop(0, CHUNKS // 2, pair, 0)
    if CHUNKS % 2:
        pltpu.make_async_copy(tab_hbm.at[g0], rows0, sem0).wait()
        pltpu.sync_copy(rows0, shared.at[d0], add=True)
    plsc.subcore_barrier()
    pltpu.sync_copy(shared.at[pl.ds(sid * ROWS_T, ROWS_T)],
                    out_hbm.at[pl.ds(cid * NPAD + sid * ROWS_T, ROWS_T)])


@functools.cache
def _get_edge_call():
    return pl.kernel(
        _edge_body,
        out_type=jax.ShapeDtypeStruct((2 * NPAD, CP), jnp.float32),
        mesh=plsc.VectorSubcoreMesh(core_axis_name="c", subcore_axis_name="s"),
        scratch_types=[
            pltpu.VMEM((CHUNKS, K), jnp.int32),
            pltpu.VMEM((K,), jnp.int32),
            pltpu.VMEM((K,), jnp.int32),
            pltpu.VMEM((K,), jnp.int32),
            pltpu.VMEM((K,), jnp.int32),
            pltpu.VMEM((K, CP), jnp.float32),
            pltpu.VMEM((K, CP), jnp.float32),
            pltpu.VMEM_SHARED((NPAD, CP), jnp.float32),
            pltpu.SemaphoreType.DMA,
            pltpu.SemaphoreType.DMA,
        ],
        compiler_params=pltpu.CompilerParams(use_tc_tiling_on_sc=False),
    )


def _msgtable_body(h_ref, e_ref, out_ref):
    a = pl.program_id(0)
    out_ref[...] = jnp.maximum(h_ref[...] + e_ref[pl.ds(a, 1), :], 0.0)


def _msgtable(h, eemb_pad):
    return pl.pallas_call(
        _msgtable_body,
        grid=(4,),
        in_specs=[pl.BlockSpec((N, CP), lambda a: (0, 0)),
                  pl.BlockSpec((4, CP), lambda a: (0, 0))],
        out_specs=pl.BlockSpec((N, CP), lambda a: (a, 0)),
        out_shape=jax.ShapeDtypeStruct((4 * N, CP), jnp.float32),
    )(h, eemb_pad)


def _embed_body(x_ref, pe_ref, nemb_ref, plw_ref, plb_ref, pbg_ref, pbb_ref,
                out_ref):
    x = x_ref[...]
    pe = pe_ref[...]
    onehot = (x == lax.broadcasted_iota(jnp.int32, (N, 28), 1)
              ).astype(jnp.float32)
    emb = jnp.dot(onehot, nemb_ref[...],
                  preferred_element_type=jnp.float32)
    m = jnp.mean(pe, axis=0, keepdims=True)
    v = jnp.mean((pe - m) * (pe - m), axis=0, keepdims=True)
    pe_n = (pe - m) * lax.rsqrt(v + 1e-5) * pbg_ref[...] + pbb_ref[...]
    pe_p = jnp.dot(pe_n, plw_ref[...],
                   preferred_element_type=jnp.float32) + plb_ref[...]
    zeros = jnp.zeros((N, CP - C), jnp.float32)
    out_ref[...] = jnp.concatenate([emb, pe_p, zeros], axis=1)


def _embed(x, pe, node_emb, pe_lin_w, pe_lin_b, pe_bn_g, pe_bn_b):
    return pl.pallas_call(
        _embed_body,
        out_shape=jax.ShapeDtypeStruct((N, CP), jnp.float32),
    )(x.reshape(N, 1).astype(jnp.int32), pe, node_emb,
      pe_lin_w, pe_lin_b.reshape(1, PE),
      pe_bn_g.reshape(1, PE), pe_bn_b.reshape(1, PE))


def _bn(x, g, b):
    m = jnp.mean(x, axis=0, keepdims=True)
    v = jnp.mean((x - m) * (x - m), axis=0, keepdims=True)
    return (x - m) * lax.rsqrt(v + 1e-5) * g + b


def _layer_body(h_ref, p0_ref, p1_ref,
                gw1_ref, gb1_ref, gw2_ref, gb2_ref,
                wq_ref, wk_ref, wv_ref, wo_ref,
                bq_ref, bk_ref, bv_ref, bo_ref,
                b1g_ref, b1b_ref, b2g_ref, b2b_ref, b3g_ref, b3b_ref,
                mw1_ref, mb1_ref, mw2_ref, mb2_ref,
                out_ref, ao_scr):
    h = h_ref[...][:, :C]
    aggr = (p0_ref[...][:N, :C] + p1_ref[...][:N, :C])

    t = h + aggr
    t = jnp.maximum(jnp.dot(t, gw1_ref[...],
                            preferred_element_type=jnp.float32)
                    + gb1_ref[...], 0.0)
    t = jnp.dot(t, gw2_ref[...],
                preferred_element_type=jnp.float32) + gb2_ref[...]
    hl = _bn(t + h, b1g_ref[...], b1b_ref[...])

    inv_sqrt = 1.0 / math.sqrt(float(DH))

    def graph(g, _):
        base = g * NPG
        hg = h_ref[pl.ds(base, NPG), :][:, :C]
        acc = jnp.zeros((NPG, C), jnp.float32)
        for hh in range(H):
            qg = jnp.dot(hg, wq_ref[hh],
                         preferred_element_type=jnp.float32) + bq_ref[hh]
            kg = jnp.dot(hg, wk_ref[hh],
                         preferred_element_type=jnp.float32) + bk_ref[hh]
            vg = jnp.dot(hg, wv_ref[hh],
                         preferred_element_type=jnp.float32) + bv_ref[hh]
            s = lax.dot_general(qg, kg, (((1,), (1,)), ((), ())),
                                preferred_element_type=jnp.float32)
            s = s * inv_sqrt
            s = s - jnp.max(s, axis=1, keepdims=True)
            p = jnp.exp(s)
            p = p / jnp.sum(p, axis=1, keepdims=True)
            og = jnp.dot(p, vg, preferred_element_type=jnp.float32)
            acc = acc + jnp.dot(og, wo_ref[hh],
                                preferred_element_type=jnp.float32)
        ao_scr[pl.ds(base, NPG), :] = acc
        return 0

    lax.fori_loop(0, G, graph, 0)

    ao = ao_scr[...] + bo_ref[...]
    ha = _bn(ao + h, b2g_ref[...], b2b_ref[...])

    out = hl + ha
    ff = jnp.maximum(jnp.dot(out, mw1_ref[...],
                             preferred_element_type=jnp.float32)
                     + mb1_ref[...], 0.0)
    ff = jnp.dot(ff, mw2_ref[...],
                 preferred_element_type=jnp.float32) + mb2_ref[...]
    hn = _bn(out + ff, b3g_ref[...], b3b_ref[...])
    out_ref[...] = jnp.concatenate(
        [hn, jnp.zeros((N, CP - C), jnp.float32)], axis=1)


def _layer(h, parts, w):
    return pl.pallas_call(
        _layer_body,
        out_shape=jax.ShapeDtypeStruct((N, CP), jnp.float32),
        scratch_shapes=[
            pltpu.VMEM((N, C), jnp.float32),
        ],
    )(h, parts[:NPAD], parts[NPAD:], *w)


def _readout_body(h1_ref, h2_ref, ow_ref, ob_ref, out_ref):
    seg = (lax.broadcasted_iota(jnp.int32, (G, N), 1) // NPG
           == lax.broadcasted_iota(jnp.int32, (G, N), 0)
           ).astype(jnp.float32)
    g1 = jnp.dot(seg, h1_ref[...][:, :C],
                 preferred_element_type=jnp.float32)
    g2 = jnp.dot(seg, h2_ref[...][:, :C],
                 preferred_element_type=jnp.float32)
    g1 = jnp.dot(g1, ow_ref[...],
                 preferred_element_type=jnp.float32) + ob_ref[...]
    g2 = jnp.dot(g2, ow_ref[...],
                 preferred_element_type=jnp.float32) + ob_ref[...]
    n1 = jnp.maximum(jnp.sqrt(jnp.sum(g1 * g1, axis=1, keepdims=True)), 1e-8)
    n2 = jnp.maximum(jnp.sqrt(jnp.sum(g2 * g2, axis=1, keepdims=True)), 1e-8)
    dot = jnp.sum(g1 * g2, axis=1, keepdims=True)
    out_ref[...] = dot / (n1 * n2)


def _readout(h1, h2, out_w, out_b):
    r = pl.pallas_call(
        _readout_body,
        out_shape=jax.ShapeDtypeStruct((G, 1), jnp.float32),
    )(h1, h2, out_w, out_b.reshape(1, 64))
    return r.reshape(G)


def kernel(x1, pe1, edge_index1, edge_attr1, batch1,
           x2, pe2, edge_index2, edge_attr2, batch2,
           node_emb, pe_lin_w, pe_lin_b, pe_bn_g, pe_bn_b, edge_emb,
           gine_w1, gine_b1, gine_w2, gine_b2,
           wq, wk, wv, wo, bq, bk, bv, bo,
           bn1_g, bn1_b, bn2_g, bn2_b, bn3_g, bn3_b,
           mlp_w1, mlp_b1, mlp_w2, mlp_b2, out_w, out_b):
    L = 2
    eemb_pad = jnp.concatenate(
        [edge_emb, jnp.zeros((4, CP - C), jnp.float32)], axis=1)

    def edge_pack(ei, ea):
        src = ei[0].astype(jnp.int32)
        dst = ei[1].astype(jnp.int32)
        eav = ea.astype(jnp.int32)
        gidx = eav * N + src
        return (gidx | (dst << 16)).reshape(NW, CHUNKS, K)

    pk1 = edge_pack(edge_index1, edge_attr1)
    pk2 = edge_pack(edge_index2, edge_attr2)

    def layer_weights(l):
        def heads(wm, bm):
            return (wm[l].reshape(C, H, DH).transpose(1, 0, 2),
                    bm[l].reshape(H, 1, DH))
        wqh, bqh = heads(wq, bq)
        wkh, bkh = heads(wk, bk)
        wvh, bvh = heads(wv, bv)
        woh = wo[l].reshape(H, DH, C)
        return (gine_w1[l], gine_b1[l].reshape(1, C),
                gine_w2[l], gine_b2[l].reshape(1, C),
                wqh, wkh, wvh, woh,
                bqh, bkh, bvh, bo[l].reshape(1, C),
                bn1_g[l].reshape(1, C), bn1_b[l].reshape(1, C),
                bn2_g[l].reshape(1, C), bn2_b[l].reshape(1, C),
                bn3_g[l].reshape(1, C), bn3_b[l].reshape(1, C),
                mlp_w1[l], mlp_b1[l].reshape(1, 2 * C),
                mlp_w2[l], mlp_b2[l].reshape(1, C))

    h1 = _embed(x1, pe1, node_emb, pe_lin_w, pe_lin_b, pe_bn_g, pe_bn_b)
    h2 = _embed(x2, pe2, node_emb, pe_lin_w, pe_lin_b, pe_bn_g, pe_bn_b)

    edge_call = _get_edge_call()
    for l in range(L):
        w = layer_weights(l)
        tab1 = _msgtable(h1, eemb_pad)
        parts1 = edge_call(tab1, pk1)
        tab2 = _msgtable(h2, eemb_pad)
        parts2 = edge_call(tab2, pk2)
        h1 = _layer(h1, parts1, w)
        h2 = _layer(h2, parts2, w)

    return _readout(h1, h2, out_w, out_b)


# speedup vs baseline: 16.4269x; 1.1575x over previous
import functools
import math

import jax
import jax.numpy as jnp
from jax import lax
from jax.experimental import pallas as pl
from jax.experimental.pallas import tpu as pltpu
from jax.experimental.pallas import tpu_sc as plsc

C = 72
CP = 128
PE = 8
H = 4
DH = 18
G = 50
NPG = 200
N = G * NPG
NPAD = 10240
E = 320000
NW = 32
K = 80
CHUNKS = (E // NW) // K
ROWS_T = NPAD // 16
ZR = 80


def _edge_body(tab_hbm, pk_hbm, out_hbm,
               pk_v, g0, d0, g1, d1, rows0, rows1, shared,
               sem0, sem1):
    cid = lax.axis_index("c")
    sid = lax.axis_index("s")
    wid = sid * 2 + cid

    def zrow(i, _):
        for j in range(CP // 16):
            rows0[i, pl.ds(j * 16, 16)] = jnp.zeros((16,), jnp.float32)
        return 0
    lax.fori_loop(0, ZR, zrow, 0)
    for z in range(ROWS_T // ZR):
        pltpu.sync_copy(rows0.at[pl.ds(0, ZR)],
                        shared.at[pl.ds(sid * ROWS_T + z * ZR, ZR)])
    plsc.subcore_barrier()

    pltpu.sync_copy(pk_hbm.at[wid], pk_v)

    def unpack(c, gbuf, dbuf):
        for j in range(K // 16):
            sl = pl.ds(j * 16, 16)
            w = pk_v[c, sl]
            gbuf[sl] = w & 0xFFFF
            dbuf[sl] = (w >> 16) & 0x3FFF

    unpack(0, g0, d0)
    pltpu.async_copy(tab_hbm.at[g0], rows0, sem0)

    def pair(i, _):
        g = i * 2
        unpack(g + 1, g1, d1)
        pltpu.async_copy(tab_hbm.at[g1], rows1, sem1)
        pltpu.make_async_copy(tab_hbm.at[g0], rows0, sem0).wait()
        pltpu.sync_copy(rows0, shared.at[d0], add=True)

        @pl.when(g + 2 < CHUNKS)
        def _():
            unpack(g + 2, g0, d0)
            pltpu.async_copy(tab_hbm.at[g0], rows0, sem0)

        pltpu.make_async_copy(tab_hbm.at[g1], rows1, sem1).wait()
        pltpu.sync_copy(rows1, shared.at[d1], add=True)
        return 0

    lax.fori_loop(0, CHUNKS // 2, pair, 0)
    if CHUNKS % 2:
        pltpu.make_async_copy(tab_hbm.at[g0], rows0, sem0).wait()
        pltpu.sync_copy(rows0, shared.at[d0], add=True)
    plsc.subcore_barrier()
    pltpu.sync_copy(shared.at[pl.ds(sid * ROWS_T, ROWS_T)],
                    out_hbm.at[pl.ds(cid * NPAD + sid * ROWS_T, ROWS_T)])


@functools.cache
def _get_edge_call():
    return pl.kernel(
        _edge_body,
        out_type=jax.ShapeDtypeStruct((2 * NPAD, CP), jnp.float32),
        mesh=plsc.VectorSubcoreMesh(core_axis_name="c", subcore_axis_name="s"),
        scratch_types=[
            pltpu.VMEM((CHUNKS, K), jnp.int32),
            pltpu.VMEM((K,), jnp.int32),
            pltpu.VMEM((K,), jnp.int32),
            pltpu.VMEM((K,), jnp.int32),
            pltpu.VMEM((K,), jnp.int32),
            pltpu.VMEM((K, CP), jnp.float32),
            pltpu.VMEM((K, CP), jnp.float32),
            pltpu.VMEM_SHARED((NPAD, CP), jnp.float32),
            pltpu.SemaphoreType.DMA,
            pltpu.SemaphoreType.DMA,
        ],
        compiler_params=pltpu.CompilerParams(use_tc_tiling_on_sc=False),
    )


def _msgtable_body(h_ref, e_ref, out_ref):
    a = pl.program_id(0)
    out_ref[...] = jnp.maximum(h_ref[...] + e_ref[pl.ds(a, 1), :], 0.0)


def _msgtable(h, eemb_pad):
    return pl.pallas_call(
        _msgtable_body,
        grid=(4,),
        in_specs=[pl.BlockSpec((N, CP), lambda a: (0, 0)),
                  pl.BlockSpec((4, CP), lambda a: (0, 0))],
        out_specs=pl.BlockSpec((N, CP), lambda a: (a, 0)),
        out_shape=jax.ShapeDtypeStruct((4 * N, CP), jnp.float32),
    )(h, eemb_pad)


def _embed_body(x_ref, pe_ref, nemb_ref, plw_ref, plb_ref, pbg_ref, pbb_ref,
                out_ref):
    x = x_ref[...]
    pe = pe_ref[...]
    onehot = (x == lax.broadcasted_iota(jnp.int32, (N, 28), 1)
              ).astype(jnp.float32)
    emb = jnp.dot(onehot, nemb_ref[...],
                  preferred_element_type=jnp.float32)
    m = jnp.mean(pe, axis=0, keepdims=True)
    v = jnp.mean((pe - m) * (pe - m), axis=0, keepdims=True)
    pe_n = (pe - m) * lax.rsqrt(v + 1e-5) * pbg_ref[...] + pbb_ref[...]
    pe_p = jnp.dot(pe_n, plw_ref[...],
                   preferred_element_type=jnp.float32) + plb_ref[...]
    zeros = jnp.zeros((N, CP - C), jnp.float32)
    out_ref[...] = jnp.concatenate([emb, pe_p, zeros], axis=1)


def _embed(x, pe, node_emb, pe_lin_w, pe_lin_b, pe_bn_g, pe_bn_b):
    return pl.pallas_call(
        _embed_body,
        out_shape=jax.ShapeDtypeStruct((N, CP), jnp.float32),
    )(x.reshape(N, 1).astype(jnp.int32), pe, node_emb,
      pe_lin_w, pe_lin_b.reshape(1, PE),
      pe_bn_g.reshape(1, PE), pe_bn_b.reshape(1, PE))


def _bn(x, g, b):
    m = jnp.mean(x, axis=0, keepdims=True)
    v = jnp.mean((x - m) * (x - m), axis=0, keepdims=True)
    return (x - m) * lax.rsqrt(v + 1e-5) * g + b


def _layer_body(h_ref, p0_ref, p1_ref,
                gw1_ref, gb1_ref, gw2_ref, gb2_ref,
                wq_ref, wk_ref, wv_ref, wo_ref,
                bq_ref, bk_ref, bv_ref, bo_ref,
                b1g_ref, b1b_ref, b2g_ref, b2b_ref, b3g_ref, b3b_ref,
                mw1_ref, mb1_ref, mw2_ref, mb2_ref,
                out_ref, ao_scr, q_scr, k_scr, v_scr):
    h = h_ref[...][:, :C]
    aggr = (p0_ref[...][:N, :C] + p1_ref[...][:N, :C])

    t = h + aggr
    t = jnp.maximum(jnp.dot(t, gw1_ref[...],
                            preferred_element_type=jnp.float32)
                    + gb1_ref[...], 0.0)
    t = jnp.dot(t, gw2_ref[...],
                preferred_element_type=jnp.float32) + gb2_ref[...]
    hl = _bn(t + h, b1g_ref[...], b1b_ref[...])

    inv_sqrt = 1.0 / math.sqrt(float(DH))

    q = jnp.dot(h, wq_ref[...],
                preferred_element_type=jnp.float32) + bq_ref[...]
    k = jnp.dot(h, wk_ref[...],
                preferred_element_type=jnp.float32) + bk_ref[...]
    v = jnp.dot(h, wv_ref[...],
                preferred_element_type=jnp.float32) + bv_ref[...]
    q_scr[...] = q
    k_scr[...] = k
    v_scr[...] = v

    col = lax.broadcasted_iota(jnp.int32, (1, C), 1) // DH
    masks = [(col == hh).astype(jnp.float32) for hh in range(H)]

    def graph(g, _):
        base = g * NPG
        qg = q_scr[pl.ds(base, NPG), :]
        kg = k_scr[pl.ds(base, NPG), :]
        vg = v_scr[pl.ds(base, NPG), :]
        kz = jnp.concatenate([kg * m for m in masks], axis=0)
        vz = jnp.concatenate([vg * m for m in masks], axis=0)
        s = lax.dot_general(qg, kz, (((1,), (1,)), ((), ())),
                            preferred_element_type=jnp.float32) * inv_sqrt
        ps = []
        for hh in range(H):
            sb = s[:, hh * NPG:(hh + 1) * NPG]
            sb = sb - jnp.max(sb, axis=1, keepdims=True)
            e = jnp.exp(sb)
            ps.append(e / jnp.sum(e, axis=1, keepdims=True))
        p = jnp.concatenate(ps, axis=1)
        ao_scr[pl.ds(base, NPG), :] = jnp.dot(
            p, vz, preferred_element_type=jnp.float32)
        return 0

    lax.fori_loop(0, G, graph, 0)

    ao = jnp.dot(ao_scr[...], wo_ref[...],
                 preferred_element_type=jnp.float32) + bo_ref[...]
    ha = _bn(ao + h, b2g_ref[...], b2b_ref[...])

    out = hl + ha
    ff = jnp.maximum(jnp.dot(out, mw1_ref[...],
                             preferred_element_type=jnp.float32)
                     + mb1_ref[...], 0.0)
    ff = jnp.dot(ff, mw2_ref[...],
                 preferred_element_type=jnp.float32) + mb2_ref[...]
    hn = _bn(out + ff, b3g_ref[...], b3b_ref[...])
    out_ref[...] = jnp.concatenate(
        [hn, jnp.zeros((N, CP - C), jnp.float32)], axis=1)


def _layer(h, parts, w):
    return pl.pallas_call(
        _layer_body,
        out_shape=jax.ShapeDtypeStruct((N, CP), jnp.float32),
        scratch_shapes=[
            pltpu.VMEM((N, C), jnp.float32),
            pltpu.VMEM((N, C), jnp.float32),
            pltpu.VMEM((N, C), jnp.float32),
            pltpu.VMEM((N, C), jnp.float32),
        ],
    )(h, parts[:NPAD], parts[NPAD:], *w)


def _readout_body(h1_ref, h2_ref, ow_ref, ob_ref, out_ref):
    seg = (lax.broadcasted_iota(jnp.int32, (G, N), 1) // NPG
           == lax.broadcasted_iota(jnp.int32, (G, N), 0)
           ).astype(jnp.float32)
    g1 = jnp.dot(seg, h1_ref[...][:, :C],
                 preferred_element_type=jnp.float32)
    g2 = jnp.dot(seg, h2_ref[...][:, :C],
                 preferred_element_type=jnp.float32)
    g1 = jnp.dot(g1, ow_ref[...],
                 preferred_element_type=jnp.float32) + ob_ref[...]
    g2 = jnp.dot(g2, ow_ref[...],
                 preferred_element_type=jnp.float32) + ob_ref[...]
    n1 = jnp.maximum(jnp.sqrt(jnp.sum(g1 * g1, axis=1, keepdims=True)), 1e-8)
    n2 = jnp.maximum(jnp.sqrt(jnp.sum(g2 * g2, axis=1, keepdims=True)), 1e-8)
    dot = jnp.sum(g1 * g2, axis=1, keepdims=True)
    out_ref[...] = dot / (n1 * n2)


def _readout(h1, h2, out_w, out_b):
    r = pl.pallas_call(
        _readout_body,
        out_shape=jax.ShapeDtypeStruct((G, 1), jnp.float32),
    )(h1, h2, out_w, out_b.reshape(1, 64))
    return r.reshape(G)


def kernel(x1, pe1, edge_index1, edge_attr1, batch1,
           x2, pe2, edge_index2, edge_attr2, batch2,
           node_emb, pe_lin_w, pe_lin_b, pe_bn_g, pe_bn_b, edge_emb,
           gine_w1, gine_b1, gine_w2, gine_b2,
           wq, wk, wv, wo, bq, bk, bv, bo,
           bn1_g, bn1_b, bn2_g, bn2_b, bn3_g, bn3_b,
           mlp_w1, mlp_b1, mlp_w2, mlp_b2, out_w, out_b):
    L = 2
    eemb_pad = jnp.concatenate(
        [edge_emb, jnp.zeros((4, CP - C), jnp.float32)], axis=1)

    def edge_pack(ei, ea):
        src = ei[0].astype(jnp.int32)
        dst = ei[1].astype(jnp.int32)
        eav = ea.astype(jnp.int32)
        gidx = eav * N + src
        return (gidx | (dst << 16)).reshape(NW, CHUNKS, K)

    pk1 = edge_pack(edge_index1, edge_attr1)
    pk2 = edge_pack(edge_index2, edge_attr2)

    def layer_weights(l):
        return (gine_w1[l], gine_b1[l].reshape(1, C),
                gine_w2[l], gine_b2[l].reshape(1, C),
                wq[l], wk[l], wv[l], wo[l],
                bq[l].reshape(1, C), bk[l].reshape(1, C),
                bv[l].reshape(1, C), bo[l].reshape(1, C),
                bn1_g[l].reshape(1, C), bn1_b[l].reshape(1, C),
                bn2_g[l].reshape(1, C), bn2_b[l].reshape(1, C),
                bn3_g[l].reshape(1, C), bn3_b[l].reshape(1, C),
                mlp_w1[l], mlp_b1[l].reshape(1, 2 * C),
                mlp_w2[l], mlp_b2[l].reshape(1, C))

    h1 = _embed(x1, pe1, node_emb, pe_lin_w, pe_lin_b, pe_bn_g, pe_bn_b)
    h2 = _embed(x2, pe2, node_emb, pe_lin_w, pe_lin_b, pe_bn_g, pe_bn_b)

    edge_call = _get_edge_call()
    for l in range(L):
        w = layer_weights(l)
        tab1 = _msgtable(h1, eemb_pad)
        parts1 = edge_call(tab1, pk1)
        tab2 = _msgtable(h2, eemb_pad)
        parts2 = edge_call(tab2, pk2)
        h1 = _layer(h1, parts1, w)
        h2 = _layer(h2, parts2, w)

    return _readout(h1, h2, out_w, out_b)
